# Initial kernel scaffold; baseline (speedup 1.0000x reference)
import jax
import jax.numpy as jnp
from jax.experimental import pallas as pl


def kernel(emb, W1, b1, W2, b2, ln_g, ln_b, Wp, bp, codex):
    raise NotImplementedError("write your pallas kernel here")


# speedup vs baseline: 1.2969x; 1.2969x over previous
import functools

import jax
import jax.numpy as jnp
from jax import lax
from jax.experimental import pallas as pl
from jax.experimental.pallas import tpu as pltpu
from jax.experimental.pallas import tpu_sc as plsc

B, T, D_MODEL, D_CODEX, K = 16, 576, 768, 64, 8192
N = B * T


def _l2norm(x):
    n = jnp.sqrt(jnp.sum(x * x, axis=-1, keepdims=True))
    return x / jnp.maximum(n, 1e-12)


def _argmax_body(emb_ref, w1_ref, b1_ref, w2_ref, b2_ref, codex_ref, idx_ref):
    bf = jnp.bfloat16
    x = emb_ref[...].astype(bf)
    h = jnp.tanh(
        lax.dot_general(x, w1_ref[...].astype(bf), (((1,), (0,)), ((), ())),
                        preferred_element_type=jnp.float32)
        + b1_ref[...])
    z = lax.dot_general(h.astype(bf), w2_ref[...].astype(bf),
                        (((1,), (0,)), ((), ())),
                        preferred_element_type=jnp.float32) + b2_ref[...]
    zn = _l2norm(z)
    cn = _l2norm(codex_ref[...])
    sim = lax.dot_general(zn.astype(bf), cn.astype(bf),
                          (((1,), (1,)), ((), ())),
                          preferred_element_type=jnp.float32)
    m = jnp.max(sim, axis=-1, keepdims=True)
    iota = lax.broadcasted_iota(jnp.int32, sim.shape, 1)
    idx_ref[...] = jnp.min(jnp.where(sim == m, iota, K), axis=-1)


def _compute_idx(emb2d, W1, b1, W2, b2, codex):
    M = 512
    grid = (N // M,)
    return pl.pallas_call(
        _argmax_body,
        grid=grid,
        in_specs=[
            pl.BlockSpec((M, D_MODEL), lambda i: (i, 0)),
            pl.BlockSpec((D_MODEL, D_MODEL), lambda i: (0, 0)),
            pl.BlockSpec((1, D_MODEL), lambda i: (0, 0)),
            pl.BlockSpec((D_MODEL, D_CODEX), lambda i: (0, 0)),
            pl.BlockSpec((1, D_CODEX), lambda i: (0, 0)),
            pl.BlockSpec((K, D_CODEX), lambda i: (0, 0)),
        ],
        out_specs=pl.BlockSpec((M,), lambda i: (i,)),
        out_shape=jax.ShapeDtypeStruct((N,), jnp.int32),
    )(emb2d, W1, b1.reshape(1, -1), W2, b2.reshape(1, -1), codex)


_NC, _NS = 2, 16
_NW = _NC * _NS
_B_PER_W = N // _NW
_CHUNK = 96
_NCHUNK = _B_PER_W // _CHUNK
_DPAD = 128


def _gather_body(codex_hbm, idx_hbm, out_hbm, idx_v, rows_v, sem):
    wid = lax.axis_index("s") * _NC + lax.axis_index("c")
    base = wid * _B_PER_W
    for j in range(_NCHUNK):
        off = base + j * _CHUNK
        pltpu.sync_copy(idx_hbm.at[pl.ds(off, _CHUNK)], idx_v)
        pltpu.async_copy(codex_hbm.at[idx_v], rows_v, sem).wait()
        pltpu.sync_copy(rows_v, out_hbm.at[pl.ds(off, _CHUNK)])


@functools.cache
def _make_gather():
    return pl.kernel(
        _gather_body,
        out_type=jax.ShapeDtypeStruct((N, _DPAD), jnp.float32),
        mesh=plsc.VectorSubcoreMesh(core_axis_name="c", subcore_axis_name="s"),
        scratch_types=[
            pltpu.VMEM((_CHUNK,), jnp.int32),
            pltpu.VMEM((_CHUNK, _DPAD), jnp.float32),
            pltpu.SemaphoreType.DMA,
        ],
    )


def _gather_rows(codex, idx):
    return _make_gather()(codex, idx)


def _post_body(q_ref, lng_ref, lnb_ref, wp_ref, bp_ref, out_ref):
    qn = _l2norm(q_ref[...])
    mu = jnp.mean(qn, axis=-1, keepdims=True)
    var = jnp.mean((qn - mu) ** 2, axis=-1, keepdims=True)
    y = (qn - mu) / jnp.sqrt(var + 1e-5) * lng_ref[...] + lnb_ref[...]
    out_ref[...] = lax.dot_general(
        y, wp_ref[...], (((1,), (0,)), ((), ())),
        preferred_element_type=jnp.float32) + bp_ref[...]


def _post_proj(zq, ln_g, ln_b, Wp, bp):
    M2 = 1024
    grid = (N // M2,)
    return pl.pallas_call(
        _post_body,
        grid=grid,
        in_specs=[
            pl.BlockSpec((M2, D_CODEX), lambda i: (i, 0)),
            pl.BlockSpec((1, D_CODEX), lambda i: (0, 0)),
            pl.BlockSpec((1, D_CODEX), lambda i: (0, 0)),
            pl.BlockSpec((D_CODEX, D_MODEL), lambda i: (0, 0)),
            pl.BlockSpec((1, D_MODEL), lambda i: (0, 0)),
        ],
        out_specs=pl.BlockSpec((M2, D_MODEL), lambda i: (i, 0)),
        out_shape=jax.ShapeDtypeStruct((N, D_MODEL), jnp.float32),
    )(zq, ln_g.reshape(1, -1), ln_b.reshape(1, -1), Wp, bp.reshape(1, -1))


def kernel(emb, W1, b1, W2, b2, ln_g, ln_b, Wp, bp, codex):
    emb2d = emb.reshape(N, D_MODEL)
    idx = _compute_idx(emb2d, W1, b1, W2, b2, codex)
    codex_pad = jnp.pad(codex, ((0, 0), (0, _DPAD - D_CODEX)))
    zq = _gather_rows(codex_pad, idx)[:, :D_CODEX]
    out = _post_proj(zq, ln_g, ln_b, Wp, bp)
    return out.reshape(B, T, D_MODEL)
